# Initial kernel scaffold; baseline (speedup 1.0000x reference)
import jax
import jax.numpy as jnp
from jax.experimental import pallas as pl


def kernel(x, ce_c1w, ce_c1b, ce_c2w, ce_c2b, ce_fw, ce_fb, se_c1w, se_c1b, se_c2w, se_c2b, se_fw, se_fb, mc_keys, mc_vals, ms_keys, ms_vals, dec_fcw, dec_fcb, dec_fsw, dec_fsb, dec_t1w, dec_t1b, dec_c1w, dec_c1b, dec_c2w, dec_c2b, dec_t2w, dec_t2b, dec_c3w, dec_c3b):
    raise NotImplementedError("write your pallas kernel here")


# speedup vs baseline: 2.8349x; 2.8349x over previous
import functools

import jax
import jax.numpy as jnp
from jax import lax
from jax.experimental import pallas as pl
from jax.experimental.pallas import tpu as pltpu
from jax.experimental.pallas import tpu_sc as plsc

B = 4096
LAT = 128
MEM = 100000
TOPK = 10

CHUNK = 128
MBLK = 2048
NMB = 49
MPAD = NMB * MBLK
NCHUNK = MPAD // CHUNK
CPB = MBLK // CHUNK
CAND = 16
BT = 512
NBT = B // BT
NEG = -1e30

NW = 32


def _knorm_body(k_ref, kn_ref):
    k = k_ref[...]
    n = jnp.sqrt(jnp.sum(k * k, axis=1, keepdims=True))
    kn_ref[...] = k / jnp.maximum(n, 1e-12)


def _knorm(keys_pad):
    return pl.pallas_call(
        _knorm_body,
        grid=(MPAD // MBLK,),
        in_specs=[pl.BlockSpec((MBLK, LAT), lambda j: (j, 0))],
        out_specs=pl.BlockSpec((MBLK, LAT), lambda j: (j, 0)),
        out_shape=jax.ShapeDtypeStruct((MPAD, LAT), jnp.float32),
    )(keys_pad)


def _sim_body(q_ref, kn_ref, sim_ref, cmax_ref):
    j = pl.program_id(1)
    q = q_ref[...]
    qn = q / jnp.maximum(jnp.sqrt(jnp.sum(q * q, axis=1, keepdims=True)), 1e-12)
    kn = kn_ref[...]
    sim = lax.dot_general(qn, kn, (((1,), (1,)), ((), ())),
                          preferred_element_type=jnp.float32)
    @pl.when(j == NMB - 1)
    def _():
        col = j * MBLK + lax.broadcasted_iota(jnp.int32, (BT, MBLK), 1)
        sim_ref[...] = jnp.where(col < MEM, sim, NEG)

    @pl.when(j != NMB - 1)
    def _():
        sim_ref[...] = sim

    s = sim_ref[...]
    parts = [jnp.max(s[:, c * CHUNK:(c + 1) * CHUNK], axis=1, keepdims=True)
             for c in range(CPB)]
    cmax_ref[...] = jnp.concatenate(parts, axis=1)[None]


def _sim_chunkmax(q, kn):
    return pl.pallas_call(
        _sim_body,
        grid=(NBT, NMB),
        in_specs=[
            pl.BlockSpec((BT, LAT), lambda i, j: (i, 0)),
            pl.BlockSpec((MBLK, LAT), lambda i, j: (j, 0)),
        ],
        out_specs=[
            pl.BlockSpec((BT, MBLK), lambda i, j: (i, j)),
            pl.BlockSpec((1, BT, CPB), lambda i, j: (j, i, 0)),
        ],
        out_shape=[
            jax.ShapeDtypeStruct((B, MPAD), jnp.float32),
            jax.ShapeDtypeStruct((NMB, B, CPB), jnp.float32),
        ],
        compiler_params=pltpu.CompilerParams(
            dimension_semantics=("arbitrary", "arbitrary")),
    )(q, kn)


def _chunksel_body(cmax_ref, cidg_ref):
    i = pl.program_id(0)
    cm = cmax_ref[...]
    ids = lax.broadcasted_iota(jnp.int32, cm.shape, 1)
    row = i * BT + lax.broadcasted_iota(jnp.int32, (BT, 1), 0)
    cols = []
    for _ in range(CAND):
        m = jnp.max(cm, axis=1, keepdims=True)
        idx = jnp.min(jnp.where(cm == m, ids, NCHUNK), axis=1, keepdims=True)
        cols.append(idx)
        cm = jnp.where(ids == idx, NEG, cm)
    c = jnp.concatenate(cols, axis=1)
    cidg_ref[...] = c + row * NCHUNK


def _chunksel(cmax):
    return pl.pallas_call(
        _chunksel_body,
        grid=(NBT,),
        in_specs=[pl.BlockSpec((BT, NCHUNK), lambda i: (i, 0))],
        out_specs=pl.BlockSpec((BT, CAND), lambda i: (i, 0)),
        out_shape=jax.ShapeDtypeStruct((B, CAND), jnp.int32),
    )(cmax)


_G_K = B * CAND
_G_IDXW = _G_K // NW
_G_SUB = 256
_G_NSUB = _G_IDXW // _G_SUB


def _sc_row_gather(table, idx_flat):
    mesh = plsc.VectorSubcoreMesh(core_axis_name="c", subcore_axis_name="s")

    @functools.partial(
        pl.kernel,
        out_type=jax.ShapeDtypeStruct((_G_K, CHUNK), jnp.float32),
        mesh=mesh,
        scratch_types=[
            pltpu.VMEM((_G_SUB,), jnp.int32),
            pltpu.VMEM((_G_SUB, CHUNK), jnp.float32),
            pltpu.SemaphoreType.DMA,
        ],
    )
    def k(table_hbm, idx_hbm, out_hbm, idx_v, buf_v, sem):
        nc = plsc.get_sparse_core_info().num_cores
        wid = lax.axis_index("s") * nc + lax.axis_index("c")
        base = wid * _G_IDXW

        def body(s, carry):
            off = base + s * _G_SUB
            pltpu.sync_copy(idx_hbm.at[pl.ds(off, _G_SUB)], idx_v)
            pltpu.async_copy(table_hbm.at[idx_v], buf_v, sem).wait()
            pltpu.sync_copy(buf_v, out_hbm.at[pl.ds(off, _G_SUB)])
            return carry

        lax.fori_loop(0, _G_NSUB, body, 0)

    return k(table, idx_flat)


def _topk_body(simg_ref, cidg_ref, w_ref, ti_ref):
    s = simg_ref[...]
    g = cidg_ref[...]
    c_local = g % NCHUNK
    ids = lax.broadcasted_iota(jnp.int32, s.shape, 1)
    cid16 = lax.broadcasted_iota(jnp.int32, (BT, CAND), 1)
    tvs, tis = [], []
    cur = s
    for _ in range(TOPK):
        m = jnp.max(cur, axis=1, keepdims=True)
        q = jnp.min(jnp.where(cur == m, ids, CAND * CHUNK),
                    axis=1, keepdims=True)
        qc = q // CHUNK
        ql = q % CHUNK
        csel = jnp.sum(jnp.where(cid16 == qc, c_local, 0),
                       axis=1, keepdims=True)
        tis.append(csel * CHUNK + ql)
        tvs.append(m)
        cur = jnp.where(ids == q, NEG, cur)
    tv = jnp.concatenate(tvs, axis=1)
    ti = jnp.concatenate(tis, axis=1)
    e = jnp.exp(tv - tv[:, 0:1])
    w = e / jnp.sum(e, axis=1, keepdims=True)
    pad_w = jnp.zeros((BT, CAND - TOPK), jnp.float32)
    pad_i = jnp.zeros((BT, CAND - TOPK), jnp.int32)
    w_ref[...] = jnp.concatenate([w, pad_w], axis=1)
    ti_ref[...] = jnp.minimum(jnp.concatenate([ti, pad_i], axis=1), MEM - 1)


def _topk_softmax(simg, cidg):
    return pl.pallas_call(
        _topk_body,
        grid=(NBT,),
        in_specs=[
            pl.BlockSpec((BT, CAND * CHUNK), lambda i: (i, 0)),
            pl.BlockSpec((BT, CAND), lambda i: (i, 0)),
        ],
        out_specs=[
            pl.BlockSpec((BT, CAND), lambda i: (i, 0)),
            pl.BlockSpec((BT, CAND), lambda i: (i, 0)),
        ],
        out_shape=[
            jax.ShapeDtypeStruct((B, CAND), jnp.float32),
            jax.ShapeDtypeStruct((B, CAND), jnp.int32),
        ],
    )(simg, cidg)


def _wcombine_body(rows_ref, w_ref, out_ref):
    rows = rows_ref[...]
    w = w_ref[...]
    out_ref[...] = jnp.sum(rows * w[..., None], axis=1)


def _weighted_combine(vals, ti_flat, w):
    rows = _sc_row_gather(vals, ti_flat).reshape(B, CAND, LAT)
    return pl.pallas_call(
        _wcombine_body,
        grid=(NBT,),
        in_specs=[
            pl.BlockSpec((BT, CAND, LAT), lambda i: (i, 0, 0)),
            pl.BlockSpec((BT, CAND), lambda i: (i, 0)),
        ],
        out_specs=pl.BlockSpec((BT, LAT), lambda i: (i, 0)),
        out_shape=jax.ShapeDtypeStruct((B, LAT), jnp.float32),
    )(rows, w)


def _memory_match(q, keys, vals):
    keys_pad = jnp.pad(keys, ((0, MPAD - MEM), (0, 0)))
    kn = _knorm(keys_pad)
    sim, cmax3 = _sim_chunkmax(q, kn)
    cmax = jnp.transpose(cmax3, (1, 0, 2)).reshape(B, NCHUNK)
    cidg = _chunksel(cmax)
    sim_table = sim.reshape(B * NCHUNK, CHUNK)
    simg = _sc_row_gather(sim_table, cidg.reshape(-1))
    w, ti = _topk_softmax(simg.reshape(B, CAND * CHUNK), cidg)
    return _weighted_combine(vals, ti.reshape(-1), w)


def _conv(x, w, b, pad=1):
    y = lax.conv_general_dilated(x, w, (1, 1), ((pad, pad), (pad, pad)),
                                 dimension_numbers=("NCHW", "OIHW", "NCHW"))
    return y + b[None, :, None, None]


def _convT(x, w, b, k=4, stride=2, pad=1):
    wf = jnp.flip(w, (2, 3)).transpose(1, 0, 2, 3)
    p = k - 1 - pad
    y = lax.conv_general_dilated(x, wf, (1, 1), ((p, p), (p, p)),
                                 lhs_dilation=(stride, stride),
                                 dimension_numbers=("NCHW", "OIHW", "NCHW"))
    return y + b[None, :, None, None]


def _enc(x, c1w, c1b, c2w, c2b, fw, fb):
    h = jax.nn.relu(_conv(x, c1w, c1b))
    h = jax.nn.relu(_conv(h, c2w, c2b))
    h = h.mean(axis=(2, 3))
    return h @ fw.T + fb


def kernel(x, ce_c1w, ce_c1b, ce_c2w, ce_c2b, ce_fw, ce_fb,
           se_c1w, se_c1b, se_c2w, se_c2b, se_fw, se_fb,
           mc_keys, mc_vals, ms_keys, ms_vals,
           dec_fcw, dec_fcb, dec_fsw, dec_fsb,
           dec_t1w, dec_t1b, dec_c1w, dec_c1b,
           dec_c2w, dec_c2b, dec_t2w, dec_t2b,
           dec_c3w, dec_c3b):
    x_center = x[:, :, 6:22, 6:22]
    z_center = _enc(x_center, ce_c1w, ce_c1b, ce_c2w, ce_c2b, ce_fw, ce_fb)
    z_match_center = _memory_match(z_center, mc_keys, mc_vals)
    z_skip = _enc(x, se_c1w, se_c1b, se_c2w, se_c2b, se_fw, se_fb)
    z_match_skip = _memory_match(z_skip, ms_keys, ms_vals)
    d_center = (z_match_center @ dec_fcw.T + dec_fcb).reshape(-1, 64, 7, 7)
    d_skip = (z_match_skip @ dec_fsw.T + dec_fsb).reshape(-1, 12, 7, 7)
    d = jnp.concatenate([d_center, d_skip], axis=1)
    h = jax.nn.relu(_convT(d, dec_t1w, dec_t1b))
    h = jax.nn.relu(_conv(h, dec_c1w, dec_c1b))
    h = jax.nn.relu(_conv(h, dec_c2w, dec_c2b))
    h = jax.nn.relu(_convT(h, dec_t2w, dec_t2b))
    return _conv(h, dec_c3w, dec_c3b)


# speedup vs baseline: 3.3589x; 1.1848x over previous
import functools

import jax
import jax.numpy as jnp
from jax import lax
from jax.experimental import pallas as pl
from jax.experimental.pallas import tpu as pltpu
from jax.experimental.pallas import tpu_sc as plsc

B = 4096
LAT = 128
MEM = 100000
TOPK = 10

CHUNK = 128
MBLK = 2048
NMB = 49
MPAD = NMB * MBLK
NCHUNK = MPAD // CHUNK
CPB = MBLK // CHUNK
CAND = 12
BT = 512
NBT = B // BT
NEG = -1e30

NW = 32


def _knorm_body(k_ref, kn_ref):
    k = k_ref[...]
    n = jnp.sqrt(jnp.sum(k * k, axis=1, keepdims=True))
    kn_ref[...] = k / jnp.maximum(n, 1e-12)


def _knorm(keys_pad):
    return pl.pallas_call(
        _knorm_body,
        grid=(MPAD // MBLK,),
        in_specs=[pl.BlockSpec((MBLK, LAT), lambda j: (j, 0))],
        out_specs=pl.BlockSpec((MBLK, LAT), lambda j: (j, 0)),
        out_shape=jax.ShapeDtypeStruct((MPAD, LAT), jnp.float32),
    )(keys_pad)


def _sim_body(q_ref, kn_ref, sim_ref, cmax_ref):
    j = pl.program_id(1)
    q = q_ref[...]
    qn = q / jnp.maximum(jnp.sqrt(jnp.sum(q * q, axis=1, keepdims=True)), 1e-12)
    kn = kn_ref[...]
    sim = lax.dot_general(qn, kn, (((1,), (1,)), ((), ())),
                          preferred_element_type=jnp.float32)

    def _emit(s):
        sim_ref[...] = s
        parts = [jnp.max(s[:, c * CHUNK:(c + 1) * CHUNK], axis=1,
                         keepdims=True) for c in range(CPB)]
        cmax_ref[...] = jnp.concatenate(parts, axis=1)[None]

    @pl.when(j == NMB - 1)
    def _():
        col = j * MBLK + lax.broadcasted_iota(jnp.int32, (BT, MBLK), 1)
        _emit(jnp.where(col < MEM, sim, NEG))

    @pl.when(j != NMB - 1)
    def _():
        _emit(sim)


def _sim_chunkmax(q, kn):
    return pl.pallas_call(
        _sim_body,
        grid=(NBT, NMB),
        in_specs=[
            pl.BlockSpec((BT, LAT), lambda i, j: (i, 0)),
            pl.BlockSpec((MBLK, LAT), lambda i, j: (j, 0)),
        ],
        out_specs=[
            pl.BlockSpec((BT, MBLK), lambda i, j: (i, j)),
            pl.BlockSpec((1, BT, CPB), lambda i, j: (j, i, 0)),
        ],
        out_shape=[
            jax.ShapeDtypeStruct((B, MPAD), jnp.float32),
            jax.ShapeDtypeStruct((NMB, B, CPB), jnp.float32),
        ],
        compiler_params=pltpu.CompilerParams(
            dimension_semantics=("arbitrary", "arbitrary")),
    )(q, kn)


def _chunksel_body(cmax_ref, cidg_ref):
    i = pl.program_id(0)
    cm = cmax_ref[...]
    ids = lax.broadcasted_iota(jnp.int32, cm.shape, 1)
    row = i * BT + lax.broadcasted_iota(jnp.int32, (BT, 1), 0)
    cols = []
    for _ in range(CAND):
        m = jnp.max(cm, axis=1, keepdims=True)
        idx = jnp.min(jnp.where(cm == m, ids, NCHUNK), axis=1, keepdims=True)
        cols.append(idx)
        cm = jnp.where(ids == idx, NEG, cm)
    c = jnp.concatenate(cols, axis=1)
    cidg_ref[...] = c + row * NCHUNK


def _chunksel(cmax):
    return pl.pallas_call(
        _chunksel_body,
        grid=(NBT,),
        in_specs=[pl.BlockSpec((BT, NCHUNK), lambda i: (i, 0))],
        out_specs=pl.BlockSpec((BT, CAND), lambda i: (i, 0)),
        out_shape=jax.ShapeDtypeStruct((B, CAND), jnp.int32),
    )(cmax)


_G_K = B * CAND
_G_IDXW = _G_K // NW
_G_SUB = 256
_G_NSUB = _G_IDXW // _G_SUB


def _sc_row_gather(table, idx_flat):
    mesh = plsc.VectorSubcoreMesh(core_axis_name="c", subcore_axis_name="s")

    @functools.partial(
        pl.kernel,
        out_type=jax.ShapeDtypeStruct((_G_K, CHUNK), jnp.float32),
        mesh=mesh,
        scratch_types=[
            pltpu.VMEM((_G_SUB,), jnp.int32),
            pltpu.VMEM((_G_SUB,), jnp.int32),
            pltpu.VMEM((_G_SUB, CHUNK), jnp.float32),
            pltpu.VMEM((_G_SUB, CHUNK), jnp.float32),
            pltpu.SemaphoreType.DMA,
            pltpu.SemaphoreType.DMA,
            pltpu.SemaphoreType.DMA,
            pltpu.SemaphoreType.DMA,
        ],
    )
    def k(table_hbm, idx_hbm, out_hbm, i0, i1, b0, b1, g0, g1, w0, w1):
        nc = plsc.get_sparse_core_info().num_cores
        wid = lax.axis_index("s") * nc + lax.axis_index("c")
        base = wid * _G_IDXW
        idxs = [i0, i1]
        bufs = [b0, b1]
        gsems = [g0, g1]
        wsems = [w0, w1]
        gh, wh = {}, {}

        def fire(s):
            slot = s % 2
            off = base + s * _G_SUB
            pltpu.sync_copy(idx_hbm.at[pl.ds(off, _G_SUB)], idxs[slot])
            gh[s] = pltpu.async_copy(table_hbm.at[idxs[slot]],
                                     bufs[slot], gsems[slot])

        fire(0)
        for s in range(_G_NSUB):
            slot = s % 2
            if s + 1 < _G_NSUB:
                if s - 1 >= 0:
                    wh[s - 1].wait()
                fire(s + 1)
            gh[s].wait()
            wh[s] = pltpu.async_copy(
                bufs[slot], out_hbm.at[pl.ds(base + s * _G_SUB, _G_SUB)],
                wsems[slot])
        if _G_NSUB >= 2:
            wh[_G_NSUB - 2].wait()
        wh[_G_NSUB - 1].wait()

    return k(table, idx_flat)


def _topk_body(simg_ref, cidg_ref, w_ref, ti_ref):
    s = simg_ref[...]
    g = cidg_ref[...]
    c_local = g % NCHUNK
    ids = lax.broadcasted_iota(jnp.int32, s.shape, 1)
    cid16 = lax.broadcasted_iota(jnp.int32, (BT, CAND), 1)
    tvs, tis = [], []
    cur = s
    for _ in range(TOPK):
        m = jnp.max(cur, axis=1, keepdims=True)
        q = jnp.min(jnp.where(cur == m, ids, CAND * CHUNK),
                    axis=1, keepdims=True)
        qc = q // CHUNK
        ql = q % CHUNK
        csel = jnp.sum(jnp.where(cid16 == qc, c_local, 0),
                       axis=1, keepdims=True)
        tis.append(csel * CHUNK + ql)
        tvs.append(m)
        cur = jnp.where(ids == q, NEG, cur)
    tv = jnp.concatenate(tvs, axis=1)
    ti = jnp.concatenate(tis, axis=1)
    e = jnp.exp(tv - tv[:, 0:1])
    w = e / jnp.sum(e, axis=1, keepdims=True)
    pad_w = jnp.zeros((BT, CAND - TOPK), jnp.float32)
    pad_i = jnp.zeros((BT, CAND - TOPK), jnp.int32)
    w_ref[...] = jnp.concatenate([w, pad_w], axis=1)
    ti_ref[...] = jnp.minimum(jnp.concatenate([ti, pad_i], axis=1), MEM - 1)


def _topk_softmax(simg, cidg):
    return pl.pallas_call(
        _topk_body,
        grid=(NBT,),
        in_specs=[
            pl.BlockSpec((BT, CAND * CHUNK), lambda i: (i, 0)),
            pl.BlockSpec((BT, CAND), lambda i: (i, 0)),
        ],
        out_specs=[
            pl.BlockSpec((BT, CAND), lambda i: (i, 0)),
            pl.BlockSpec((BT, CAND), lambda i: (i, 0)),
        ],
        out_shape=[
            jax.ShapeDtypeStruct((B, CAND), jnp.float32),
            jax.ShapeDtypeStruct((B, CAND), jnp.int32),
        ],
    )(simg, cidg)


def _wcombine_body(rows_ref, w_ref, out_ref):
    rows = rows_ref[...]
    w = w_ref[...]
    out_ref[...] = jnp.sum(rows * w[..., None], axis=1)


def _weighted_combine(vals, ti_flat, w):
    rows = _sc_row_gather(vals, ti_flat).reshape(B, CAND, LAT)
    return pl.pallas_call(
        _wcombine_body,
        grid=(NBT,),
        in_specs=[
            pl.BlockSpec((BT, CAND, LAT), lambda i: (i, 0, 0)),
            pl.BlockSpec((BT, CAND), lambda i: (i, 0)),
        ],
        out_specs=pl.BlockSpec((BT, LAT), lambda i: (i, 0)),
        out_shape=jax.ShapeDtypeStruct((B, LAT), jnp.float32),
    )(rows, w)


def _memory_match_pair(qs, keys_list, vals_list):
    kns = [_knorm(jnp.pad(k, ((0, MPAD - MEM), (0, 0)))) for k in keys_list]
    sims, cmaxs = zip(*[_sim_chunkmax(q, kn) for q, kn in zip(qs, kns)])
    cidgs = [_chunksel(jnp.transpose(c3, (1, 0, 2)).reshape(B, NCHUNK))
             for c3 in cmaxs]
    simgs = [_sc_row_gather(sim.reshape(B * NCHUNK, CHUNK), cidg.reshape(-1))
             for sim, cidg in zip(sims, cidgs)]
    wtis = [_topk_softmax(simg.reshape(B, CAND * CHUNK), cidg)
            for simg, cidg in zip(simgs, cidgs)]
    return [_weighted_combine(vals, ti.reshape(-1), w)
            for vals, (w, ti) in zip(vals_list, wtis)]


def _conv(x, w, b, pad=1):
    y = lax.conv_general_dilated(x, w, (1, 1), ((pad, pad), (pad, pad)),
                                 dimension_numbers=("NCHW", "OIHW", "NCHW"))
    return y + b[None, :, None, None]


def _convT(x, w, b, k=4, stride=2, pad=1):
    wf = jnp.flip(w, (2, 3)).transpose(1, 0, 2, 3)
    p = k - 1 - pad
    y = lax.conv_general_dilated(x, wf, (1, 1), ((p, p), (p, p)),
                                 lhs_dilation=(stride, stride),
                                 dimension_numbers=("NCHW", "OIHW", "NCHW"))
    return y + b[None, :, None, None]


def _enc(x, c1w, c1b, c2w, c2b, fw, fb):
    h = jax.nn.relu(_conv(x, c1w, c1b))
    h = jax.nn.relu(_conv(h, c2w, c2b))
    h = h.mean(axis=(2, 3))
    return h @ fw.T + fb


def kernel(x, ce_c1w, ce_c1b, ce_c2w, ce_c2b, ce_fw, ce_fb,
           se_c1w, se_c1b, se_c2w, se_c2b, se_fw, se_fb,
           mc_keys, mc_vals, ms_keys, ms_vals,
           dec_fcw, dec_fcb, dec_fsw, dec_fsb,
           dec_t1w, dec_t1b, dec_c1w, dec_c1b,
           dec_c2w, dec_c2b, dec_t2w, dec_t2b,
           dec_c3w, dec_c3b):
    x_center = x[:, :, 6:22, 6:22]
    z_center = _enc(x_center, ce_c1w, ce_c1b, ce_c2w, ce_c2b, ce_fw, ce_fb)
    z_skip = _enc(x, se_c1w, se_c1b, se_c2w, se_c2b, se_fw, se_fb)
    z_match_center, z_match_skip = _memory_match_pair(
        [z_center, z_skip], [mc_keys, ms_keys], [mc_vals, ms_vals])
    d_center = (z_match_center @ dec_fcw.T + dec_fcb).reshape(-1, 64, 7, 7)
    d_skip = (z_match_skip @ dec_fsw.T + dec_fsb).reshape(-1, 12, 7, 7)
    d = jnp.concatenate([d_center, d_skip], axis=1)
    h = jax.nn.relu(_convT(d, dec_t1w, dec_t1b))
    h = jax.nn.relu(_conv(h, dec_c1w, dec_c1b))
    h = jax.nn.relu(_conv(h, dec_c2w, dec_c2b))
    h = jax.nn.relu(_convT(h, dec_t2w, dec_t2b))
    return _conv(h, dec_c3w, dec_c3b)
